# Initial kernel scaffold; baseline (speedup 1.0000x reference)
import jax
import jax.numpy as jnp
from jax.experimental import pallas as pl


def kernel(x, edge_index, edge_attr, params):
    raise NotImplementedError("write your pallas kernel here")


# speedup vs baseline: 2.8936x; 2.8936x over previous
import functools

import jax
import jax.numpy as jnp
from jax import lax
from jax.experimental import pallas as pl
from jax.experimental.pallas import tpu as pltpu
from jax.experimental.pallas import tpu_sc as plsc

N_NODES = 10000
N_EDGES = 320000
DIM = 128
EDIM = 16
NUM_LAYERS = 3

NC = 2
NS = 16
NW = NC * NS
EPW = N_EDGES // NW
CHUNK = 80
NCHUNKS = EPW // CHUNK
N_PAD = 10240
RPW = N_PAD // NS
ZROWS = 128


def _sc_body(h_hbm, e_hbm, src_hbm, dst_hbm, out_hbm,
             idx_src, idx_dst, rows, ev, zbuf, aggr, gsem):
  cid = lax.axis_index("c")
  sid = lax.axis_index("s")
  wid = sid * NC + cid
  wbase = wid * EPW

  def zrow(i, _):
    for j in range(DIM // 16):
      zbuf[i, pl.ds(j * 16, 16)] = jnp.zeros((16,), jnp.float32)
    return 0
  lax.fori_loop(0, ZROWS, zrow, 0)
  for k in range(RPW // ZROWS):
    pltpu.sync_copy(zbuf, aggr.at[pl.ds(sid * RPW + k * ZROWS, ZROWS)])
  plsc.subcore_barrier()

  def chunk(ci, _):
    base = wbase + ci * CHUNK
    pltpu.sync_copy(src_hbm.at[pl.ds(base, CHUNK)], idx_src)
    pltpu.sync_copy(dst_hbm.at[pl.ds(base, CHUNK)], idx_dst)
    cp = pltpu.async_copy(h_hbm.at[idx_src], rows, gsem)
    pltpu.sync_copy(e_hbm.at[pl.ds(base, CHUNK)], ev)
    cp.wait()

    def row(i, _):
      for j in range(DIM // 16):
        sl = pl.ds(j * 16, 16)
        rows[i, sl] = jnp.maximum(rows[i, sl] + ev[i, sl], 0.0)
      return 0
    lax.fori_loop(0, CHUNK, row, 0)

    pltpu.sync_copy(rows, aggr.at[idx_dst], add=True)
    return 0
  lax.fori_loop(0, NCHUNKS, chunk, 0)

  plsc.subcore_barrier()
  pltpu.sync_copy(aggr.at[pl.ds(sid * RPW, RPW)],
                  out_hbm.at[cid, pl.ds(sid * RPW, RPW)])


_sc_aggregate = pl.kernel(
    _sc_body,
    out_type=jax.ShapeDtypeStruct((NC, N_PAD, DIM), jnp.float32),
    mesh=plsc.VectorSubcoreMesh(core_axis_name="c", subcore_axis_name="s",
                                num_cores=NC, num_subcores=NS),
    scratch_types=[
        pltpu.VMEM((CHUNK,), jnp.int32),
        pltpu.VMEM((CHUNK,), jnp.int32),
        pltpu.VMEM((CHUNK, DIM), jnp.float32),
        pltpu.VMEM((CHUNK, DIM), jnp.float32),
        pltpu.VMEM((ZROWS, DIM), jnp.float32),
        pltpu.VMEM_SHARED((N_PAD, DIM), jnp.float32),
        pltpu.SemaphoreType.DMA,
    ],
)


def _edge_mm_body(ea_ref, w_ref, b_ref, o1_ref, o2_ref, o3_ref):
  ea = ea_ref[...]
  for i, o_ref in enumerate((o1_ref, o2_ref, o3_ref)):
    o_ref[...] = jnp.dot(ea, w_ref[i], preferred_element_type=jnp.float32) \
        + b_ref[i]


def _edge_transforms(edge_attr, w_stack, b_stack):
  be = 4000
  grid = (N_EDGES // be,)
  out = jax.ShapeDtypeStruct((N_EDGES, DIM), jnp.float32)
  return pl.pallas_call(
      _edge_mm_body,
      grid=grid,
      in_specs=[
          pl.BlockSpec((be, EDIM), lambda i: (i, 0)),
          pl.BlockSpec((NUM_LAYERS, EDIM, DIM), lambda i: (0, 0, 0)),
          pl.BlockSpec((NUM_LAYERS, 1, DIM), lambda i: (0, 0, 0)),
      ],
      out_specs=[pl.BlockSpec((be, DIM), lambda i: (i, 0))] * 3,
      out_shape=[out, out, out],
  )(edge_attr, w_stack, b_stack)


def _mlp_body(h_ref, p0_ref, p1_ref, w1_ref, b1_ref, w2_ref, b2_ref, o_ref):
  a = h_ref[...] + p0_ref[...] + p1_ref[...]
  h1 = jnp.maximum(
      jnp.dot(a, w1_ref[...], preferred_element_type=jnp.float32)
      + b1_ref[...], 0.0)
  o_ref[...] = jnp.maximum(
      jnp.dot(h1, w2_ref[...], preferred_element_type=jnp.float32)
      + b2_ref[...], 0.0)


def _mlp(h, p0, p1, w1, b1, w2, b2):
  bn = 2000
  grid = (N_NODES // bn,)
  return pl.pallas_call(
      _mlp_body,
      grid=grid,
      in_specs=[
          pl.BlockSpec((bn, DIM), lambda i: (i, 0)),
          pl.BlockSpec((bn, DIM), lambda i: (i, 0)),
          pl.BlockSpec((bn, DIM), lambda i: (i, 0)),
          pl.BlockSpec((DIM, DIM), lambda i: (0, 0)),
          pl.BlockSpec((1, DIM), lambda i: (0, 0)),
          pl.BlockSpec((DIM, DIM), lambda i: (0, 0)),
          pl.BlockSpec((1, DIM), lambda i: (0, 0)),
      ],
      out_specs=pl.BlockSpec((bn, DIM), lambda i: (i, 0)),
      out_shape=jax.ShapeDtypeStruct((N_NODES, DIM), jnp.float32),
  )(h, p0, p1, w1, b1, w2, b2)


@jax.jit
def _run(x, edge_index, edge_attr, params):
  src = edge_index[0]
  dst = edge_index[1]

  w_stack = jnp.stack([p[0] for p in params])
  b_stack = jnp.stack([p[1][None, :] for p in params])
  e_all = _edge_transforms(edge_attr, w_stack, b_stack)

  h = x
  for l in range(NUM_LAYERS):
    _, _, w1, b1, w2, b2 = params[l]
    parts = _sc_aggregate(h, e_all[l], src, dst)
    h = _mlp(h, parts[0, :N_NODES], parts[1, :N_NODES],
             w1, b1[None, :], w2, b2[None, :])
  return h


def kernel(x, edge_index, edge_attr, params):
  return _run(x, edge_index, edge_attr, params)


# speedup vs baseline: 3.7615x; 1.2999x over previous
import functools

import jax
import jax.numpy as jnp
from jax import lax
from jax.experimental import pallas as pl
from jax.experimental.pallas import tpu as pltpu
from jax.experimental.pallas import tpu_sc as plsc

N_NODES = 10000
N_EDGES = 320000
DIM = 128
EDIM = 16
NUM_LAYERS = 3

NC = 2
NS = 16
NW = NC * NS
EPW = N_EDGES // NW
CHUNK = 40
NCHUNKS = EPW // CHUNK
N_PAD = 10240
RPW = N_PAD // NS
ZROWS = 16


def _sc_body(h_hbm, e_hbm, sd_hbm, out_hbm,
             sd0, sd1, rows0, rows1, ev0, ev1, zbuf, aggr,
             isem0, isem1, dsem0, dsem1):
  cid = lax.axis_index("c")
  sid = lax.axis_index("s")
  wid = sid * NC + cid
  wbase = wid * EPW

  def zrow(i, _):
    for j in range(DIM // 16):
      zbuf[i, pl.ds(j * 16, 16)] = jnp.zeros((16,), jnp.float32)
    return 0
  lax.fori_loop(0, ZROWS, zrow, 0)
  for k in range(RPW // ZROWS):
    pltpu.sync_copy(zbuf, aggr.at[pl.ds(sid * RPW + k * ZROWS, ZROWS)])
  plsc.subcore_barrier()

  def fire_idx(ci, sdbuf, isem):
    pltpu.async_copy(sd_hbm.at[wid, ci], sdbuf, isem)

  def fire_gather(ci, sdbuf, rbuf, ebuf, isem, dsem):
    pltpu.make_async_copy(sd_hbm.at[0, 0], sdbuf, isem).wait()
    pltpu.async_copy(h_hbm.at[sdbuf.at[0]], rbuf, dsem)
    pltpu.async_copy(e_hbm.at[pl.ds(wbase + ci * CHUNK, CHUNK)], ebuf, dsem)

  def process(ci, sdbuf, rbuf, ebuf, dsem):
    pltpu.make_async_copy(e_hbm.at[pl.ds(0, CHUNK)], rbuf, dsem).wait()
    pltpu.make_async_copy(e_hbm.at[pl.ds(0, CHUNK)], ebuf, dsem).wait()

    def row(i, _):
      for j in range(DIM // 16):
        sl = pl.ds(j * 16, 16)
        rbuf[i, sl] = jnp.maximum(rbuf[i, sl] + ebuf[i, sl], 0.0)
      return 0
    lax.fori_loop(0, CHUNK, row, 0)

    pltpu.sync_copy(rbuf, aggr.at[sdbuf.at[1]], add=True)

  fire_idx(0, sd0, isem0)
  fire_idx(1, sd1, isem1)
  fire_gather(0, sd0, rows0, ev0, isem0, dsem0)

  def pair(i, _):
    c0 = 2 * i
    fire_gather(c0 + 1, sd1, rows1, ev1, isem1, dsem1)
    process(c0, sd0, rows0, ev0, dsem0)

    @pl.when(c0 + 2 < NCHUNKS)
    def _():
      fire_idx(c0 + 2, sd0, isem0)
      fire_gather(c0 + 2, sd0, rows0, ev0, isem0, dsem0)
    process(c0 + 1, sd1, rows1, ev1, dsem1)

    @pl.when(c0 + 3 < NCHUNKS)
    def _():
      fire_idx(c0 + 3, sd1, isem1)
    return 0
  lax.fori_loop(0, NCHUNKS // 2, pair, 0)

  plsc.subcore_barrier()
  pltpu.sync_copy(aggr.at[pl.ds(sid * RPW, RPW)],
                  out_hbm.at[cid, pl.ds(sid * RPW, RPW)])


_sc_aggregate = pl.kernel(
    _sc_body,
    out_type=jax.ShapeDtypeStruct((NC, N_PAD, DIM), jnp.float32),
    mesh=plsc.VectorSubcoreMesh(core_axis_name="c", subcore_axis_name="s",
                                num_cores=NC, num_subcores=NS),
    scratch_types=[
        pltpu.VMEM((2, CHUNK), jnp.int32),
        pltpu.VMEM((2, CHUNK), jnp.int32),
        pltpu.VMEM((CHUNK, DIM), jnp.float32),
        pltpu.VMEM((CHUNK, DIM), jnp.float32),
        pltpu.VMEM((CHUNK, DIM), jnp.float32),
        pltpu.VMEM((CHUNK, DIM), jnp.float32),
        pltpu.VMEM((ZROWS, DIM), jnp.float32),
        pltpu.VMEM_SHARED((N_PAD, DIM), jnp.float32),
        pltpu.SemaphoreType.DMA,
        pltpu.SemaphoreType.DMA,
        pltpu.SemaphoreType.DMA,
        pltpu.SemaphoreType.DMA,
    ],
)


def _edge_mm_body(ea_ref, w_ref, b_ref, o1_ref, o2_ref, o3_ref):
  ea = ea_ref[...]
  for i, o_ref in enumerate((o1_ref, o2_ref, o3_ref)):
    o_ref[...] = jnp.dot(ea, w_ref[i], preferred_element_type=jnp.float32) \
        + b_ref[i]


def _edge_transforms(edge_attr, w_stack, b_stack):
  be = 4000
  grid = (N_EDGES // be,)
  out = jax.ShapeDtypeStruct((N_EDGES, DIM), jnp.float32)
  return pl.pallas_call(
      _edge_mm_body,
      grid=grid,
      in_specs=[
          pl.BlockSpec((be, EDIM), lambda i: (i, 0)),
          pl.BlockSpec((NUM_LAYERS, EDIM, DIM), lambda i: (0, 0, 0)),
          pl.BlockSpec((NUM_LAYERS, 1, DIM), lambda i: (0, 0, 0)),
      ],
      out_specs=[pl.BlockSpec((be, DIM), lambda i: (i, 0))] * 3,
      out_shape=[out, out, out],
  )(edge_attr, w_stack, b_stack)


def _mlp_body(h_ref, p0_ref, p1_ref, w1_ref, b1_ref, w2_ref, b2_ref, o_ref):
  a = h_ref[...] + p0_ref[...] + p1_ref[...]
  h1 = jnp.maximum(
      jnp.dot(a, w1_ref[...], preferred_element_type=jnp.float32)
      + b1_ref[...], 0.0)
  o_ref[...] = jnp.maximum(
      jnp.dot(h1, w2_ref[...], preferred_element_type=jnp.float32)
      + b2_ref[...], 0.0)


def _mlp(h, p0, p1, w1, b1, w2, b2):
  bn = 2000
  grid = (N_NODES // bn,)
  return pl.pallas_call(
      _mlp_body,
      grid=grid,
      in_specs=[
          pl.BlockSpec((bn, DIM), lambda i: (i, 0)),
          pl.BlockSpec((bn, DIM), lambda i: (i, 0)),
          pl.BlockSpec((bn, DIM), lambda i: (i, 0)),
          pl.BlockSpec((DIM, DIM), lambda i: (0, 0)),
          pl.BlockSpec((1, DIM), lambda i: (0, 0)),
          pl.BlockSpec((DIM, DIM), lambda i: (0, 0)),
          pl.BlockSpec((1, DIM), lambda i: (0, 0)),
      ],
      out_specs=pl.BlockSpec((bn, DIM), lambda i: (i, 0)),
      out_shape=jax.ShapeDtypeStruct((N_NODES, DIM), jnp.float32),
  )(h, p0, p1, w1, b1, w2, b2)


@jax.jit
def _run(x, edge_index, edge_attr, params):
  sd = edge_index.reshape(2, NW, NCHUNKS, CHUNK).transpose(1, 2, 0, 3)

  w_stack = jnp.stack([p[0] for p in params])
  b_stack = jnp.stack([p[1][None, :] for p in params])
  e_all = _edge_transforms(edge_attr, w_stack, b_stack)

  h = x
  for l in range(NUM_LAYERS):
    _, _, w1, b1, w2, b2 = params[l]
    parts = _sc_aggregate(h, e_all[l], sd)
    h = _mlp(h, parts[0, :N_NODES], parts[1, :N_NODES],
             w1, b1[None, :], w2, b2[None, :])
  return h


def kernel(x, edge_index, edge_attr, params):
  return _run(x, edge_index, edge_attr, params)
